# Initial kernel scaffold; baseline (speedup 1.0000x reference)
import jax
import jax.numpy as jnp
from jax.experimental import pallas as pl


def kernel(df, bd, targets):
    raise NotImplementedError("write your pallas kernel here")


# speedup vs baseline: 109.1107x; 109.1107x over previous
import jax
import jax.numpy as jnp
from jax.experimental import pallas as pl

_RADIUS = 5
_H = 128
_W = 128
_IN = _H - 2 * _RADIUS

_DELTAS = [
    (dx, dy)
    for dx in range(-_RADIUS, _RADIUS + 1)
    for dy in range(-_RADIUS, _RADIUS + 1)
    if dx * dx + dy * dy < _RADIUS * _RADIUS and dx + dy != 0
]


def _loss_kernel(df0_ref, df1_ref, tg_ref, out_ref):
    r = _RADIUS
    f0c = df0_ref[:, r:r + _IN, r:r + _IN]
    f1c = df1_ref[:, r:r + _IN, r:r + _IN]
    tc = tg_ref[:, r:r + _IN, r:r + _IN] > 0

    fg_sum = jnp.float32(0.0)
    bg_sum = jnp.float32(0.0)
    fg_cnt = jnp.float32(0.0)
    for dx, dy in _DELTAS:
        ys = r + dy
        xs = r + dx
        s0 = df0_ref[:, ys:ys + _IN, xs:xs + _IN]
        s1 = df1_ref[:, ys:ys + _IN, xs:xs + _IN]
        st = tg_ref[:, ys:ys + _IN, xs:xs + _IN]
        d0 = s0 - f0c
        d1 = s1 - f1c
        fgf = jnp.where(tc & (st > 0), jnp.float32(1.0), jnp.float32(0.0))
        absterm = jnp.abs(d0 - jnp.float32(dy)) + jnp.abs(d1 - jnp.float32(dx))
        fg_sum += jnp.sum(fgf * absterm)
        bg_sum += jnp.sum((1.0 - fgf) * (d0 + d1))
        fg_cnt += jnp.sum(fgf)

    total = jnp.float32(len(_DELTAS) * _IN * _IN * tg_ref.shape[0])
    bg_cnt = total - fg_cnt
    loss = (fg_sum / jnp.maximum(fg_cnt, 1.0)
            + bg_sum / jnp.maximum(bg_cnt, 1.0))
    out_ref[:, :] = loss[None, None]


def kernel(df, bd, targets):
    del bd
    B, C, h, w = df.shape
    df0 = df[:, 0]
    df1 = df[:, 1]
    out = pl.pallas_call(
        _loss_kernel,
        out_shape=jax.ShapeDtypeStruct((1, 1), jnp.float32),
    )(df0, df1, targets)
    return out[0, 0]


# speedup vs baseline: 188.6283x; 1.7288x over previous
import jax
import jax.numpy as jnp
from jax.experimental import pallas as pl
from jax.experimental.pallas import tpu as pltpu

_RADIUS = 5
_H = 128
_W = 128
_IN = _H - 2 * _RADIUS

_DELTAS = [
    (dx, dy)
    for dx in range(-_RADIUS, _RADIUS + 1)
    for dy in range(-_RADIUS, _RADIUS + 1)
    if dx * dx + dy * dy < _RADIUS * _RADIUS and dx + dy != 0
]


def _loss_kernel(df0_ref, df1_ref, tg_ref, out_ref, tp_ref):
    r = _RADIUS
    tp_ref[...] = jnp.where(tg_ref[...] > 0, jnp.float32(1.0), jnp.float32(0.0))

    f0c = df0_ref[:, r:r + _IN, r:r + _IN]
    f1c = df1_ref[:, r:r + _IN, r:r + _IN]
    tcf = tp_ref[:, r:r + _IN, r:r + _IN]

    accf = jnp.zeros((_IN, _IN), jnp.float32)
    accb = jnp.zeros((_IN, _IN), jnp.float32)
    accc = jnp.zeros((_IN, _IN), jnp.float32)
    for dx, dy in _DELTAS:
        ys = r + dy
        xs = r + dx
        s0 = df0_ref[:, ys:ys + _IN, xs:xs + _IN]
        s1 = df1_ref[:, ys:ys + _IN, xs:xs + _IN]
        sb = tp_ref[:, ys:ys + _IN, xs:xs + _IN]
        d0 = s0 - f0c
        d1 = s1 - f1c
        fgf = tcf * sb
        ab = jnp.abs(d0 - jnp.float32(dy)) + jnp.abs(d1 - jnp.float32(dx))
        s = d0 + d1
        accf = accf + jnp.sum(fgf * ab, axis=0)
        accb = accb + jnp.sum(s - fgf * s, axis=0)
        accc = accc + jnp.sum(fgf, axis=0)

    fg_sum = jnp.sum(accf)
    bg_sum = jnp.sum(accb)
    fg_cnt = jnp.sum(accc)
    total = jnp.float32(len(_DELTAS) * _IN * _IN * tg_ref.shape[0])
    bg_cnt = total - fg_cnt
    loss = (fg_sum / jnp.maximum(fg_cnt, 1.0)
            + bg_sum / jnp.maximum(bg_cnt, 1.0))
    out_ref[:, :] = loss[None, None]


def kernel(df, bd, targets):
    del bd
    df0 = df[:, 0]
    df1 = df[:, 1]
    out = pl.pallas_call(
        _loss_kernel,
        out_shape=jax.ShapeDtypeStruct((1, 1), jnp.float32),
        scratch_shapes=[pltpu.VMEM((df.shape[0], _H, _W), jnp.float32)],
    )(df0, df1, targets)
    return out[0, 0]
